# baseline (speedup 1.0000x reference)
import functools

import jax
import jax.numpy as jnp
from jax import lax
from jax.experimental import pallas as pl
from jax.experimental.pallas import tpu as pltpu
from jax.experimental.pallas import tpu_sc as plsc

B = 4096
L = 200
D = 128
V = 129
VP = 136
NW = 32
ROWS_PER_W = B // NW
TOK_PER_W = ROWS_PER_W * L
CH = 128
NCHUNK = TOK_PER_W // CH
NBUF = 5
NGROUP = NCHUNK // NBUF
LB = 8


def _build_body(table_ref, pe_ref, out_ref):
    t = table_ref[...]
    for j in range(LB):
        out_ref[pl.ds(j * VP, VP)] = t + pe_ref[j]


_build_combined = pl.pallas_call(
    _build_body,
    grid=(L // LB,),
    in_specs=[
        pl.BlockSpec((VP, D), lambda i: (0, 0)),
        pl.BlockSpec((LB, 1, D), lambda i: (i, 0, 0)),
    ],
    out_specs=pl.BlockSpec((LB * VP, D), lambda i: (i, 0)),
    out_shape=jax.ShapeDtypeStruct((L * VP, D), jnp.float32),
)


@functools.partial(
    pl.kernel,
    out_type=jax.ShapeDtypeStruct((B * L, D), jnp.float32),
    mesh=plsc.VectorSubcoreMesh(core_axis_name="c", subcore_axis_name="s"),
    scratch_types=[
        pltpu.VMEM((TOK_PER_W,), jnp.int32),
        pltpu.VMEM((L,), jnp.int32),
        pltpu.VMEM((16,), jnp.int32),
        pltpu.VMEM((CH, D), jnp.float32),
        pltpu.VMEM((CH, D), jnp.float32),
        pltpu.VMEM((CH, D), jnp.float32),
        pltpu.VMEM((CH, D), jnp.float32),
        pltpu.VMEM((CH, D), jnp.float32),
        pltpu.SemaphoreType.DMA,
        pltpu.SemaphoreType.DMA,
        pltpu.SemaphoreType.DMA,
        pltpu.SemaphoreType.DMA,
        pltpu.SemaphoreType.DMA,
        pltpu.SemaphoreType.DMA,
        pltpu.SemaphoreType.DMA,
        pltpu.SemaphoreType.DMA,
        pltpu.SemaphoreType.DMA,
        pltpu.SemaphoreType.DMA,
    ],
)
def _embed_sc(seq_hbm, comb_hbm, offs_hbm, offst_hbm, out_hbm,
              idx_all, offs_v, offst_v, buf0, buf1, buf2, buf3, buf4,
              gsem0, gsem1, gsem2, gsem3, gsem4,
              ssem0, ssem1, ssem2, ssem3, ssem4):
    wid = lax.axis_index("s") * 2 + lax.axis_index("c")
    base_tok = wid * TOK_PER_W
    bufs = (buf0, buf1, buf2, buf3, buf4)
    gsems = (gsem0, gsem1, gsem2, gsem3, gsem4)
    ssems = (ssem0, ssem1, ssem2, ssem3, ssem4)

    pltpu.sync_copy(seq_hbm.at[pl.ds(base_tok, TOK_PER_W)], idx_all)
    pltpu.sync_copy(offs_hbm, offs_v)
    pltpu.sync_copy(offst_hbm, offst_v)

    def fix_row(r, c):
        rb = r * L
        for k in range(12):
            sl = pl.ds(rb + k * 16, 16)
            idx_all[sl] = idx_all[sl] + offs_v[pl.ds(k * 16, 16)]
        sl = pl.ds(rb + 184, 16)
        idx_all[sl] = idx_all[sl] + offst_v[...]
        return c

    lax.fori_loop(0, ROWS_PER_W, fix_row, 0)

    def start_g(c, b):
        pltpu.async_copy(
            comb_hbm.at[idx_all.at[pl.ds(c * CH, CH)]], bufs[b], gsems[b]
        )

    def wait_g(c, b):
        pltpu.make_async_copy(
            comb_hbm.at[idx_all.at[pl.ds(c * CH, CH)]], bufs[b], gsems[b]
        ).wait()

    def start_s(c, b):
        pltpu.async_copy(
            bufs[b], out_hbm.at[pl.ds(base_tok + c * CH, CH)], ssems[b]
        )

    def wait_s(c, b):
        pltpu.make_async_copy(
            bufs[b], out_hbm.at[pl.ds(base_tok + c * CH, CH)], ssems[b]
        ).wait()

    for b in range(NBUF):
        start_g(b, b)

    def group(g, carry):
        c0 = NBUF * g
        for b in range(NBUF):
            wait_g(c0 + b, b)
            start_s(c0 + b, b)

        @pl.when(g < NGROUP - 1)
        def _prefetch():
            for b in range(NBUF):
                wait_s(c0 + b, b)
                start_g(c0 + NBUF + b, b)

        return carry

    lax.fori_loop(0, NGROUP, group, 0)
    for b in range(NBUF):
        wait_s(0, b)


def kernel(sequence, token_table, pe):
    pe_slice = pe[0, :L, :].reshape(L, 1, D)
    table_pad = jnp.pad(token_table, ((0, VP - V), (0, 0)))
    combined = _build_combined(table_pad, pe_slice)
    seq = sequence.reshape(B * L).astype(jnp.int32)
    offs = jnp.arange(L, dtype=jnp.int32) * VP
    offs_tail = jnp.concatenate(
        [jnp.zeros(8, jnp.int32), jnp.arange(192, 200, dtype=jnp.int32) * VP]
    )
    out = _embed_sc(seq, combined, offs, offs_tail)
    return out.reshape(B, L, D)


# speedup vs baseline: 1.0226x; 1.0226x over previous
import functools

import jax
import jax.numpy as jnp
from jax import lax
from jax.experimental import pallas as pl
from jax.experimental.pallas import tpu as pltpu
from jax.experimental.pallas import tpu_sc as plsc

B = 4096
L = 200
D = 128
V = 129
VP = 136
NW = 32
ROWS_PER_W = B // NW
TOK_PER_W = ROWS_PER_W * L
CH = 128
NCHUNK = TOK_PER_W // CH
NBUF = 5
NGROUP = NCHUNK // NBUF
LB = 40


def _build_body(table_ref, pe_ref, out_ref):
    t = table_ref[...]
    for j in range(LB):
        out_ref[pl.ds(j * VP, VP)] = t + pe_ref[j]


_build_combined = pl.pallas_call(
    _build_body,
    grid=(L // LB,),
    in_specs=[
        pl.BlockSpec((VP, D), lambda i: (0, 0)),
        pl.BlockSpec((LB, 1, D), lambda i: (i, 0, 0)),
    ],
    out_specs=pl.BlockSpec((LB * VP, D), lambda i: (i, 0)),
    out_shape=jax.ShapeDtypeStruct((L * VP, D), jnp.float32),
)


@functools.partial(
    pl.kernel,
    out_type=jax.ShapeDtypeStruct((B * L, D), jnp.float32),
    mesh=plsc.VectorSubcoreMesh(core_axis_name="c", subcore_axis_name="s"),
    scratch_types=[
        pltpu.VMEM((TOK_PER_W,), jnp.int32),
        pltpu.VMEM((L,), jnp.int32),
        pltpu.VMEM((16,), jnp.int32),
        pltpu.VMEM((CH, D), jnp.float32),
        pltpu.VMEM((CH, D), jnp.float32),
        pltpu.VMEM((CH, D), jnp.float32),
        pltpu.VMEM((CH, D), jnp.float32),
        pltpu.VMEM((CH, D), jnp.float32),
        pltpu.SemaphoreType.DMA,
        pltpu.SemaphoreType.DMA,
        pltpu.SemaphoreType.DMA,
        pltpu.SemaphoreType.DMA,
        pltpu.SemaphoreType.DMA,
        pltpu.SemaphoreType.DMA,
        pltpu.SemaphoreType.DMA,
        pltpu.SemaphoreType.DMA,
        pltpu.SemaphoreType.DMA,
        pltpu.SemaphoreType.DMA,
    ],
)
def _embed_sc(seq_hbm, comb_hbm, offs_hbm, offst_hbm, out_hbm,
              idx_all, offs_v, offst_v, buf0, buf1, buf2, buf3, buf4,
              gsem0, gsem1, gsem2, gsem3, gsem4,
              ssem0, ssem1, ssem2, ssem3, ssem4):
    wid = lax.axis_index("s") * 2 + lax.axis_index("c")
    base_tok = wid * TOK_PER_W
    bufs = (buf0, buf1, buf2, buf3, buf4)
    gsems = (gsem0, gsem1, gsem2, gsem3, gsem4)
    ssems = (ssem0, ssem1, ssem2, ssem3, ssem4)

    pltpu.sync_copy(seq_hbm.at[pl.ds(base_tok, TOK_PER_W)], idx_all)
    pltpu.sync_copy(offs_hbm, offs_v)
    pltpu.sync_copy(offst_hbm, offst_v)

    def fix_row(r, c):
        rb = r * L
        for k in range(12):
            sl = pl.ds(rb + k * 16, 16)
            idx_all[sl] = idx_all[sl] + offs_v[pl.ds(k * 16, 16)]
        sl = pl.ds(rb + 184, 16)
        idx_all[sl] = idx_all[sl] + offst_v[...]
        return c

    lax.fori_loop(0, ROWS_PER_W, fix_row, 0)

    def start_g(c, b):
        pltpu.async_copy(
            comb_hbm.at[idx_all.at[pl.ds(c * CH, CH)]], bufs[b], gsems[b]
        )

    def wait_g(c, b):
        pltpu.make_async_copy(
            comb_hbm.at[idx_all.at[pl.ds(c * CH, CH)]], bufs[b], gsems[b]
        ).wait()

    def start_s(c, b):
        pltpu.async_copy(
            bufs[b], out_hbm.at[pl.ds(base_tok + c * CH, CH)], ssems[b]
        )

    def wait_s(c, b):
        pltpu.make_async_copy(
            bufs[b], out_hbm.at[pl.ds(base_tok + c * CH, CH)], ssems[b]
        ).wait()

    for b in range(NBUF):
        start_g(b, b)

    def group(g, carry):
        c0 = NBUF * g
        for b in range(NBUF):
            wait_g(c0 + b, b)
            start_s(c0 + b, b)

        @pl.when(g < NGROUP - 1)
        def _prefetch():
            for b in range(NBUF):
                wait_s(c0 + b, b)
                start_g(c0 + NBUF + b, b)

        return carry

    lax.fori_loop(0, NGROUP, group, 0)
    for b in range(NBUF):
        wait_s(0, b)


def kernel(sequence, token_table, pe):
    pe_slice = pe[0, :L, :].reshape(L, 1, D)
    table_pad = jnp.pad(token_table, ((0, VP - V), (0, 0)))
    combined = _build_combined(table_pad, pe_slice)
    seq = sequence.reshape(B * L).astype(jnp.int32)
    offs = jnp.arange(L, dtype=jnp.int32) * VP
    offs_tail = jnp.concatenate(
        [jnp.zeros(8, jnp.int32), jnp.arange(192, 200, dtype=jnp.int32) * VP]
    )
    out = _embed_sc(seq, combined, offs, offs_tail)
    return out.reshape(B, L, D)


# speedup vs baseline: 1.0266x; 1.0039x over previous
import functools

import jax
import jax.numpy as jnp
from jax import lax
from jax.experimental import pallas as pl
from jax.experimental.pallas import tpu as pltpu
from jax.experimental.pallas import tpu_sc as plsc

B = 4096
L = 200
D = 128
V = 129
VP = 136
NW = 32
ROWS_PER_W = B // NW
TOK_PER_W = ROWS_PER_W * L
CH = 128
NCHUNK = TOK_PER_W // CH
NBUF = 5
NGROUP = NCHUNK // NBUF
LB = 200


def _build_body(table_ref, pe_ref, out_ref):
    t = table_ref[...]
    for j in range(LB):
        out_ref[pl.ds(j * VP, VP)] = t + pe_ref[j]


_build_combined = pl.pallas_call(
    _build_body,
    grid=(L // LB,),
    in_specs=[
        pl.BlockSpec((VP, D), lambda i: (0, 0)),
        pl.BlockSpec((LB, 1, D), lambda i: (i, 0, 0)),
    ],
    out_specs=pl.BlockSpec((LB * VP, D), lambda i: (i, 0)),
    out_shape=jax.ShapeDtypeStruct((L * VP, D), jnp.float32),
)


@functools.partial(
    pl.kernel,
    out_type=jax.ShapeDtypeStruct((B * L, D), jnp.float32),
    mesh=plsc.VectorSubcoreMesh(core_axis_name="c", subcore_axis_name="s"),
    scratch_types=[
        pltpu.VMEM((TOK_PER_W,), jnp.int32),
        pltpu.VMEM((L,), jnp.int32),
        pltpu.VMEM((16,), jnp.int32),
        pltpu.VMEM((CH, D), jnp.float32),
        pltpu.VMEM((CH, D), jnp.float32),
        pltpu.VMEM((CH, D), jnp.float32),
        pltpu.VMEM((CH, D), jnp.float32),
        pltpu.VMEM((CH, D), jnp.float32),
        pltpu.SemaphoreType.DMA,
        pltpu.SemaphoreType.DMA,
        pltpu.SemaphoreType.DMA,
        pltpu.SemaphoreType.DMA,
        pltpu.SemaphoreType.DMA,
        pltpu.SemaphoreType.DMA,
        pltpu.SemaphoreType.DMA,
        pltpu.SemaphoreType.DMA,
        pltpu.SemaphoreType.DMA,
        pltpu.SemaphoreType.DMA,
    ],
)
def _embed_sc(seq_hbm, comb_hbm, offs_hbm, offst_hbm, out_hbm,
              idx_all, offs_v, offst_v, buf0, buf1, buf2, buf3, buf4,
              gsem0, gsem1, gsem2, gsem3, gsem4,
              ssem0, ssem1, ssem2, ssem3, ssem4):
    wid = lax.axis_index("s") * 2 + lax.axis_index("c")
    base_tok = wid * TOK_PER_W
    bufs = (buf0, buf1, buf2, buf3, buf4)
    gsems = (gsem0, gsem1, gsem2, gsem3, gsem4)
    ssems = (ssem0, ssem1, ssem2, ssem3, ssem4)

    pltpu.sync_copy(seq_hbm.at[pl.ds(base_tok, TOK_PER_W)], idx_all)
    pltpu.sync_copy(offs_hbm, offs_v)
    pltpu.sync_copy(offst_hbm, offst_v)

    def fix_row(r, c):
        rb = r * L
        for k in range(12):
            sl = pl.ds(rb + k * 16, 16)
            idx_all[sl] = idx_all[sl] + offs_v[pl.ds(k * 16, 16)]
        sl = pl.ds(rb + 184, 16)
        idx_all[sl] = idx_all[sl] + offst_v[...]
        return c

    lax.fori_loop(0, ROWS_PER_W, fix_row, 0)

    def start_g(c, b):
        pltpu.async_copy(
            comb_hbm.at[idx_all.at[pl.ds(c * CH, CH)]], bufs[b], gsems[b]
        )

    def wait_g(c, b):
        pltpu.make_async_copy(
            comb_hbm.at[idx_all.at[pl.ds(c * CH, CH)]], bufs[b], gsems[b]
        ).wait()

    def start_s(c, b):
        pltpu.async_copy(
            bufs[b], out_hbm.at[pl.ds(base_tok + c * CH, CH)], ssems[b]
        )

    def wait_s(c, b):
        pltpu.make_async_copy(
            bufs[b], out_hbm.at[pl.ds(base_tok + c * CH, CH)], ssems[b]
        ).wait()

    for b in range(NBUF):
        start_g(b, b)

    def group(g, carry):
        c0 = NBUF * g
        for b in range(NBUF):
            wait_g(c0 + b, b)
            start_s(c0 + b, b)

        @pl.when(g < NGROUP - 1)
        def _prefetch():
            for b in range(NBUF):
                wait_s(c0 + b, b)
                start_g(c0 + NBUF + b, b)

        return carry

    lax.fori_loop(0, NGROUP, group, 0)
    for b in range(NBUF):
        wait_s(0, b)


def kernel(sequence, token_table, pe):
    pe_slice = pe[0, :L, :].reshape(L, 1, D)
    table_pad = jnp.pad(token_table, ((0, VP - V), (0, 0)))
    combined = _build_combined(table_pad, pe_slice)
    seq = sequence.reshape(B * L).astype(jnp.int32)
    offs = jnp.arange(L, dtype=jnp.int32) * VP
    offs_tail = jnp.concatenate(
        [jnp.zeros(8, jnp.int32), jnp.arange(192, 200, dtype=jnp.int32) * VP]
    )
    out = _embed_sc(seq, combined, offs, offs_tail)
    return out.reshape(B, L, D)


# speedup vs baseline: 1.0459x; 1.0188x over previous
import functools

import jax
import jax.numpy as jnp
from jax import lax
from jax.experimental import pallas as pl
from jax.experimental.pallas import tpu as pltpu
from jax.experimental.pallas import tpu_sc as plsc

B = 4096
L = 200
D = 128
V = 129
VP = 136
NW = 32
ROWS_PER_W = B // NW
TOK_PER_W = ROWS_PER_W * L
CH = 128
NCHUNK = TOK_PER_W // CH
NBUF = 5
NGROUP = NCHUNK // NBUF
PERIOD = 3200
NPAT = PERIOD // CH


def _build_body(table_ref, pe_ref, out_ref):
    t = table_ref[...]
    for j in range(L):
        out_ref[pl.ds(j * VP, VP)] = t + pe_ref[j]


_build_combined = pl.pallas_call(
    _build_body,
    grid=(1,),
    in_specs=[
        pl.BlockSpec((VP, D), lambda i: (0, 0)),
        pl.BlockSpec((L, 1, D), lambda i: (0, 0, 0)),
    ],
    out_specs=pl.BlockSpec((L * VP, D), lambda i: (0, 0)),
    out_shape=jax.ShapeDtypeStruct((L * VP, D), jnp.float32),
)


@functools.partial(
    pl.kernel,
    out_type=jax.ShapeDtypeStruct((B * L, D), jnp.float32),
    mesh=plsc.VectorSubcoreMesh(core_axis_name="c", subcore_axis_name="s"),
    scratch_types=[
        pltpu.VMEM((TOK_PER_W,), jnp.int32),
        pltpu.VMEM((PERIOD,), jnp.int32),
        pltpu.VMEM((CH, D), jnp.float32),
        pltpu.VMEM((CH, D), jnp.float32),
        pltpu.VMEM((CH, D), jnp.float32),
        pltpu.VMEM((CH, D), jnp.float32),
        pltpu.VMEM((CH, D), jnp.float32),
        pltpu.SemaphoreType.DMA,
        pltpu.SemaphoreType.DMA,
        pltpu.SemaphoreType.DMA,
        pltpu.SemaphoreType.DMA,
        pltpu.SemaphoreType.DMA,
        pltpu.SemaphoreType.DMA,
        pltpu.SemaphoreType.DMA,
        pltpu.SemaphoreType.DMA,
        pltpu.SemaphoreType.DMA,
        pltpu.SemaphoreType.DMA,
    ],
)
def _embed_sc(seq_hbm, comb_hbm, pat_hbm, out_hbm,
              idx_all, pat_v, buf0, buf1, buf2, buf3, buf4,
              gsem0, gsem1, gsem2, gsem3, gsem4,
              ssem0, ssem1, ssem2, ssem3, ssem4):
    wid = lax.axis_index("s") * 2 + lax.axis_index("c")
    base_tok = wid * TOK_PER_W
    bufs = (buf0, buf1, buf2, buf3, buf4)
    gsems = (gsem0, gsem1, gsem2, gsem3, gsem4)
    ssems = (ssem0, ssem1, ssem2, ssem3, ssem4)

    pltpu.sync_copy(seq_hbm.at[pl.ds(base_tok, TOK_PER_W)], idx_all)
    pltpu.sync_copy(pat_hbm, pat_v)

    def fix_chunk(c):
        pb = lax.rem(c, NPAT) * CH
        cb = c * CH
        for k in range(CH // 16):
            sl = pl.ds(cb + k * 16, 16)
            idx_all[sl] = idx_all[sl] + pat_v[pl.ds(pb + k * 16, 16)]

    def start_g(c, b):
        pltpu.async_copy(
            comb_hbm.at[idx_all.at[pl.ds(c * CH, CH)]], bufs[b], gsems[b]
        )

    def wait_g(c, b):
        pltpu.make_async_copy(
            comb_hbm.at[idx_all.at[pl.ds(c * CH, CH)]], bufs[b], gsems[b]
        ).wait()

    def start_s(c, b):
        pltpu.async_copy(
            bufs[b], out_hbm.at[pl.ds(base_tok + c * CH, CH)], ssems[b]
        )

    def wait_s(c, b):
        pltpu.make_async_copy(
            bufs[b], out_hbm.at[pl.ds(base_tok + c * CH, CH)], ssems[b]
        ).wait()

    for b in range(NBUF):
        fix_chunk(b)
        start_g(b, b)

    def group(g, carry):
        c0 = NBUF * g
        for b in range(NBUF):
            wait_g(c0 + b, b)
            start_s(c0 + b, b)

        @pl.when(g < NGROUP - 1)
        def _prefetch():
            for b in range(NBUF):
                wait_s(c0 + b, b)
                fix_chunk(c0 + NBUF + b)
                start_g(c0 + NBUF + b, b)

        return carry

    lax.fori_loop(0, NGROUP, group, 0)
    for b in range(NBUF):
        wait_s(0, b)


def kernel(sequence, token_table, pe):
    pe_slice = pe[0, :L, :].reshape(L, 1, D)
    table_pad = jnp.pad(token_table, ((0, VP - V), (0, 0)))
    combined = _build_combined(table_pad, pe_slice)
    seq = sequence.reshape(B * L).astype(jnp.int32)
    pat = (jnp.arange(PERIOD, dtype=jnp.int32) % L) * VP
    out = _embed_sc(seq, combined, pat)
    return out.reshape(B, L, D)


# speedup vs baseline: 1.0506x; 1.0046x over previous
import functools

import jax
import jax.numpy as jnp
from jax import lax
from jax.experimental import pallas as pl
from jax.experimental.pallas import tpu as pltpu
from jax.experimental.pallas import tpu_sc as plsc

B = 4096
L = 200
D = 128
V = 129
VP = 136
NW = 32
ROWS_PER_W = B // NW
TOK_PER_W = ROWS_PER_W * L
CH = 128
NCHUNK = TOK_PER_W // CH
NBUF = 5
NGROUP = NCHUNK // NBUF
PERIOD = 3200
NPAT = PERIOD // CH


def _build_body(table_ref, pe_ref, out_ref):
    t = jnp.concatenate(
        [table_ref[...], jnp.zeros((VP - V, D), jnp.float32)], axis=0
    )
    for j in range(L):
        out_ref[pl.ds(j * VP, VP)] = t + pe_ref[j]


_build_combined = pl.pallas_call(
    _build_body,
    grid=(1,),
    in_specs=[
        pl.BlockSpec((V, D), lambda i: (0, 0)),
        pl.BlockSpec((L, 1, D), lambda i: (0, 0, 0)),
    ],
    out_specs=pl.BlockSpec((L * VP, D), lambda i: (0, 0)),
    out_shape=jax.ShapeDtypeStruct((L * VP, D), jnp.float32),
)


@functools.partial(
    pl.kernel,
    out_type=jax.ShapeDtypeStruct((B * L, D), jnp.float32),
    mesh=plsc.VectorSubcoreMesh(core_axis_name="c", subcore_axis_name="s"),
    scratch_types=[
        pltpu.VMEM((TOK_PER_W,), jnp.int32),
        pltpu.VMEM((PERIOD,), jnp.int32),
        pltpu.VMEM((CH, D), jnp.float32),
        pltpu.VMEM((CH, D), jnp.float32),
        pltpu.VMEM((CH, D), jnp.float32),
        pltpu.VMEM((CH, D), jnp.float32),
        pltpu.VMEM((CH, D), jnp.float32),
        pltpu.SemaphoreType.DMA,
        pltpu.SemaphoreType.DMA,
        pltpu.SemaphoreType.DMA,
        pltpu.SemaphoreType.DMA,
        pltpu.SemaphoreType.DMA,
        pltpu.SemaphoreType.DMA,
        pltpu.SemaphoreType.DMA,
        pltpu.SemaphoreType.DMA,
        pltpu.SemaphoreType.DMA,
        pltpu.SemaphoreType.DMA,
    ],
)
def _embed_sc(seq_hbm, comb_hbm, pat_hbm, out_hbm,
              idx_all, pat_v, buf0, buf1, buf2, buf3, buf4,
              gsem0, gsem1, gsem2, gsem3, gsem4,
              ssem0, ssem1, ssem2, ssem3, ssem4):
    wid = lax.axis_index("s") * 2 + lax.axis_index("c")
    base_tok = wid * TOK_PER_W
    bufs = (buf0, buf1, buf2, buf3, buf4)
    gsems = (gsem0, gsem1, gsem2, gsem3, gsem4)
    ssems = (ssem0, ssem1, ssem2, ssem3, ssem4)

    pltpu.sync_copy(seq_hbm.at[pl.ds(base_tok, TOK_PER_W)], idx_all)
    pltpu.sync_copy(pat_hbm, pat_v)

    def fix_chunk(c):
        pb = lax.rem(c, NPAT) * CH
        cb = c * CH
        for k in range(CH // 16):
            sl = pl.ds(cb + k * 16, 16)
            idx_all[sl] = idx_all[sl] + pat_v[pl.ds(pb + k * 16, 16)]

    def start_g(c, b):
        pltpu.async_copy(
            comb_hbm.at[idx_all.at[pl.ds(c * CH, CH)]], bufs[b], gsems[b]
        )

    def wait_g(c, b):
        pltpu.make_async_copy(
            comb_hbm.at[idx_all.at[pl.ds(c * CH, CH)]], bufs[b], gsems[b]
        ).wait()

    def start_s(c, b):
        pltpu.async_copy(
            bufs[b], out_hbm.at[pl.ds(base_tok + c * CH, CH)], ssems[b]
        )

    def wait_s(c, b):
        pltpu.make_async_copy(
            bufs[b], out_hbm.at[pl.ds(base_tok + c * CH, CH)], ssems[b]
        ).wait()

    for b in range(NBUF):
        fix_chunk(b)
        start_g(b, b)

    def group(g, carry):
        c0 = NBUF * g
        for b in range(NBUF):
            wait_g(c0 + b, b)
            start_s(c0 + b, b)

        @pl.when(g < NGROUP - 1)
        def _prefetch():
            for b in range(NBUF):
                wait_s(c0 + b, b)
                fix_chunk(c0 + NBUF + b)
                start_g(c0 + NBUF + b, b)

        return carry

    lax.fori_loop(0, NGROUP, group, 0)
    for b in range(NBUF):
        wait_s(0, b)


def kernel(sequence, token_table, pe):
    pe_slice = pe[0, :L, :].reshape(L, 1, D)
    combined = _build_combined(token_table, pe_slice)
    seq = sequence.reshape(B * L).astype(jnp.int32)
    pat = (jnp.arange(PERIOD, dtype=jnp.int32) % L) * VP
    out = _embed_sc(seq, combined, pat)
    return out.reshape(B, L, D)
